# Initial kernel scaffold; baseline (speedup 1.0000x reference)
import jax
import jax.numpy as jnp
from jax.experimental import pallas as pl


def kernel(x, edge_index, u, e, spec_edge, W_se1, b_se1, W_se2, b_se2, W_sp1, b_sp1, W_sp2, b_sp2, g_ln, beta_ln, Wc1, bc1, Wc2, bc2, Wc3, bc3, Wc5, bc5, Wc6, bc6, Wc7, bc7, Wc8, bc8, Wd1, bd1, Wd2, bd2):
    raise NotImplementedError("write your pallas kernel here")


# speedup vs baseline: 1.4651x; 1.4651x over previous
import functools

import jax
import jax.numpy as jnp
from jax import lax
from jax.experimental import pallas as pl
from jax.experimental.pallas import tpu as pltpu
from jax.experimental.pallas import tpu_sc as plsc

F32 = jnp.float32
I32 = jnp.int32
NEG_INF = float("-inf")
NW = 32


def _make_segmax(n_edges, chunk, n_pad, npw, sentinel):
    assert n_edges % chunk == 0 and chunk % 128 == 0
    nrounds = n_edges // chunk
    sub = chunk // 128
    mesh = plsc.VectorSubcoreMesh(core_axis_name="c", subcore_axis_name="s")

    @functools.partial(
        pl.kernel,
        mesh=mesh,
        compiler_params=pltpu.CompilerParams(needs_layout_passes=False),
        out_type=jax.ShapeDtypeStruct((n_pad, 64), F32),
        scratch_types=[
            pltpu.VMEM((chunk,), I32),
            pltpu.VMEM((chunk,), I32),
            pltpu.VMEM((sub, 128), I32),
            pltpu.VMEM((chunk,), I32),
            pltpu.VMEM((128, 128), F32),
            pltpu.VMEM((npw, 64), F32),
            pltpu.SemaphoreType.DMA,
        ],
    )
    def segmax(src_hbm, dst_hbm, b_hbm, out_hbm, dstv, srcv, msrc, mdst, rows, acc, sem):
        wid = lax.axis_index("s") * 2 + lax.axis_index("c")
        lo = wid * npw
        hi = lo + npw
        zeros16 = jnp.zeros((16,), I32)
        sent16 = jnp.full((16,), sentinel, I32)
        ninf16 = jnp.full((16,), NEG_INF, F32)

        def init_acc(r, carry):
            for j in range(4):
                acc[r, pl.ds(j * 16, 16)] = ninf16
            return carry

        lax.fori_loop(0, npw, init_acc, 0)

        def init_lists(i, carry):
            msrc[i // 8, pl.ds((i % 8) * 16, 16)] = sent16
            mdst[pl.ds(i * 16, 16)] = zeros16
            return carry

        lax.fori_loop(0, sub * 8, init_lists, 0)

        def round_body(r, carry):
            pltpu.sync_copy(dst_hbm.at[pl.ds(r * chunk, chunk)], dstv)
            pltpu.sync_copy(src_hbm.at[pl.ds(r * chunk, chunk)], srcv)

            def filt(i, cntv):
                d = dstv[pl.ds(i * 16, 16)]
                s = srcv[pl.ds(i * 16, 16)]
                m = (d >= lo) & (d < hi)
                pfx = jnp.cumsum(m.astype(I32))
                idx = cntv + pfx - 1
                plsc.store_scatter(msrc, [idx >> 7, idx & 127], s, mask=m)
                plsc.store_scatter(mdst, [idx], d - lo, mask=m)
                return cntv + plsc.all_reduce_population_count(m)

            cntv = lax.fori_loop(0, chunk // 16, filt, zeros16)
            cnt = jnp.max(cntv)
            nsub_r = (cnt + 127) // 128

            def gath(g, carry):
                pltpu.async_copy(b_hbm.at[msrc.at[g]], rows, sem).wait()

                def accum16(t, carry2):
                    dv = mdst[pl.ds(g * 128 + t * 16, 16)]
                    for j in range(16):
                        dl = dv[j]
                        for jj in range(4):
                            sl = pl.ds(jj * 16, 16)
                            acc[dl, sl] = jnp.maximum(acc[dl, sl], rows[t * 16 + j, sl])
                    return carry2

                lax.fori_loop(0, 8, accum16, 0)
                return carry

            lax.fori_loop(0, nsub_r, gath, 0)
            return carry

        lax.fori_loop(0, nrounds, round_body, 0)
        pltpu.sync_copy(acc, out_hbm.at[pl.ds(lo, npw)])

    return segmax


def _ln(t, g, b):
    mu = jnp.mean(t, axis=-1, keepdims=True)
    v = jnp.mean((t - mu) ** 2, axis=-1, keepdims=True)
    return (t - mu) / jnp.sqrt(v + 1e-5) * g + b


def _conv(xfeat, src, dst, W, b, segfn, tab_pad):
    fdim = xfeat.shape[1]
    n = xfeat.shape[0]
    A = xfeat @ (W[:fdim] - W[fdim:]) + b
    B = xfeat @ W[fdim:]
    Bp = jnp.pad(B, ((0, tab_pad - n), (0, 0)), constant_values=NEG_INF)
    Bp = jnp.pad(Bp, ((0, 0), (0, 64)))
    seg = segfn(src, dst, Bp)[:n]
    t = A + seg
    return jnp.where(jnp.isneginf(t), 0.0, t)


def kernel(x, edge_index, u, e, spec_edge, W_se1, b_se1, W_se2, b_se2, W_sp1, b_sp1, W_sp2, b_sp2, g_ln, beta_ln, Wc1, bc1, Wc2, bc2, Wc3, bc3, Wc5, bc5, Wc6, bc6, Wc7, bc7, Wc8, bc8, Wd1, bd1, Wd2, bd2):
    lr = jax.nn.leaky_relu
    ln = lambda t: _ln(t, g_ln, beta_ln)
    seg_n = _make_segmax(160000, 6400, 10240, 320, sentinel=10000)
    seg_k = _make_segmax(1024, 1024, 256, 8, sentinel=64)
    src, dst = edge_index[0], edge_index[1]
    ssrc, sdst = spec_edge[0], spec_edge[1]
    conv_n = lambda xf, W, b: _conv(xf, src, dst, W, b, seg_n, 10240)
    conv_k = lambda sf, W, b: _conv(sf, ssrc, sdst, W, b, seg_k, 72)

    s0 = ln(lr(ln(lr(e @ W_se1 + b_se1)) @ W_se2 + b_se2))
    x0 = ln(lr(ln(lr(x @ W_sp1 + b_sp1)) @ W_sp2 + b_sp2))
    s1 = ln(lr(conv_k(s0, Wc1, bc1)))
    x1 = ln(lr(conv_n(x0, Wc5, bc5)))
    s1 = jnp.concatenate([s1, u.T @ x0], axis=1)
    x1 = jnp.concatenate([x1, u @ s0], axis=1)
    s2 = jnp.concatenate([s0, s1], axis=1)
    x2 = jnp.concatenate([x0, x1], axis=1)
    s2 = ln(lr(conv_k(s2, Wc2, bc2)))
    x2 = ln(lr(conv_n(x2, Wc6, bc6)))
    s2 = jnp.concatenate([s2, u.T @ x1], axis=1)
    x2 = jnp.concatenate([x2, u @ s1], axis=1)
    s3 = jnp.concatenate([s0, s1, s2], axis=1)
    x3 = jnp.concatenate([x0, x1, x2], axis=1)
    s3 = ln(lr(conv_k(s3, Wc3, bc3)))
    x3 = ln(lr(conv_n(x3, Wc7, bc7)))
    s3 = jnp.concatenate([s3, u.T @ x2], axis=1)
    x3 = jnp.concatenate([x3, u @ s2], axis=1)
    x4 = jnp.concatenate([x0, x1, x2, x3], axis=1)
    x4 = ln(lr(conv_n(x4, Wc8, bc8)))
    x4 = jnp.concatenate([x4, u @ s3], axis=1)
    x4 = jnp.concatenate([x0, x1, x2, x3, x4], axis=1)
    out = ln(lr(x4 @ Wd1 + bd1))
    out = out @ Wd2 + bd2
    return out
